# baseline (speedup 1.0000x reference)
import functools

import jax
import jax.numpy as jnp
from jax import lax
from jax.experimental import pallas as pl
from jax.experimental.pallas import tpu as pltpu
from jax.experimental.pallas import tpu_sc as plsc

NUM_EMBEDDINGS = 1000000
EMBED_DIM = 64
SEQ = 4096
NUM_TOKENS = 200

_INFO = plsc.get_sparse_core_info()
NC = _INFO.num_cores
NS = _INFO.num_subcores
NW = NC * NS
CHUNK = 128
B = SEQ * NUM_TOKENS
NCHUNKS = B // CHUNK
CPW = NCHUNKS // NW


def _body(idx_hbm, table_hbm, out_hbm, idx_v, rows_v, sem):
    wid = lax.axis_index("s") * NC + lax.axis_index("c")
    cbase = wid * CPW
    pltpu.sync_copy(idx_hbm.at[pl.ds(cbase, CPW)], idx_v)

    @pl.loop(0, CPW)
    def step(j):
        pltpu.async_copy(table_hbm.at[idx_v.at[j]], rows_v, sem).wait()
        pltpu.sync_copy(rows_v, out_hbm.at[pl.ds((cbase + j) * CHUNK, CHUNK)])


_sc_gather = pl.kernel(
    _body,
    out_type=jax.ShapeDtypeStruct((B, EMBED_DIM), jnp.float32),
    mesh=plsc.VectorSubcoreMesh(core_axis_name="c", subcore_axis_name="s"),
    scratch_types=[
        pltpu.VMEM((CPW, CHUNK), jnp.int32),
        pltpu.VMEM((CHUNK, EMBED_DIM), jnp.float32),
        pltpu.SemaphoreType.DMA,
    ],
    compiler_params=pltpu.CompilerParams(use_tc_tiling_on_sc=False),
)


@jax.jit
def kernel(x, embed_table):
    idx = x.astype(jnp.int32).reshape(NCHUNKS, CHUNK)
    out = _sc_gather(idx, embed_table)
    return out.reshape(SEQ, NUM_TOKENS, EMBED_DIM)


# speedup vs baseline: 1.1175x; 1.1175x over previous
import functools

import jax
import jax.numpy as jnp
from jax import lax
from jax.experimental import pallas as pl
from jax.experimental.pallas import tpu as pltpu
from jax.experimental.pallas import tpu_sc as plsc

NUM_EMBEDDINGS = 1000000
EMBED_DIM = 64
SEQ = 4096
NUM_TOKENS = 200

_INFO = plsc.get_sparse_core_info()
NC = _INFO.num_cores
NS = _INFO.num_subcores
NW = NC * NS
CHUNK = 128
B = SEQ * NUM_TOKENS
NCHUNKS = B // CHUNK
CPW = NCHUNKS // NW


GROUP = 2
NBUF = 4
GROWS = GROUP * CHUNK
NG = CPW // GROUP


def _body(idx_hbm, table_hbm, out_hbm, idx_v,
          buf0, buf1, buf2, buf3,
          g0, g1, g2, g3, s0, s1, s2, s3):
    bufs = (buf0, buf1, buf2, buf3)
    gsems = (g0, g1, g2, g3)
    ssems = (s0, s1, s2, s3)

    wid = lax.axis_index("s") * NC + lax.axis_index("c")
    cbase = wid * CPW
    rbase = cbase * CHUNK
    pltpu.sync_copy(idx_hbm.at[pl.ds(cbase, CPW)], idx_v)

    def fire_gather(grp, buf, gsem):
        for c in range(GROUP):
            pltpu.async_copy(table_hbm.at[idx_v.at[grp * GROUP + c]],
                             buf.at[pl.ds(c * CHUNK, CHUNK)], gsem)

    def drain(buf, sem):
        pltpu.make_async_copy(table_hbm.at[pl.ds(0, GROWS)], buf, sem).wait()

    for b in range(NBUF):
        fire_gather(b, bufs[b], gsems[b])

    @pl.loop(0, NG, step=NBUF)
    def cycle(g):
        for b in range(NBUF):
            grp = g + b
            drain(bufs[b], gsems[b])
            pltpu.async_copy(
                bufs[b], out_hbm.at[pl.ds(rbase + grp * GROWS, GROWS)],
                ssems[b])
            drain(bufs[b], ssems[b])

            @pl.when(grp + NBUF < NG)
            def _():
                fire_gather(grp + NBUF, bufs[b], gsems[b])


_sc_gather = pl.kernel(
    _body,
    out_type=jax.ShapeDtypeStruct((B, EMBED_DIM), jnp.float32),
    mesh=plsc.VectorSubcoreMesh(core_axis_name="c", subcore_axis_name="s"),
    scratch_types=(
        [pltpu.VMEM((CPW, CHUNK), jnp.int32)]
        + [pltpu.VMEM((GROWS, EMBED_DIM), jnp.float32)] * NBUF
        + [pltpu.SemaphoreType.DMA] * (2 * NBUF)
    ),
    compiler_params=pltpu.CompilerParams(use_tc_tiling_on_sc=False),
)


@jax.jit
def kernel(x, embed_table):
    idx = x.astype(jnp.int32).reshape(NCHUNKS, CHUNK)
    out = _sc_gather(idx, embed_table)
    return out.reshape(SEQ, NUM_TOKENS, EMBED_DIM)
